# Initial kernel scaffold; baseline (speedup 1.0000x reference)
import jax
import jax.numpy as jnp
from jax.experimental import pallas as pl


def kernel(edge_index, node_features, edgetypes, W_enc, b_enc, rel_weight, loop_weight, h_bias):
    raise NotImplementedError("write your pallas kernel here")


# speedup vs baseline: 14.9453x; 14.9453x over previous
import functools

import jax
import jax.numpy as jnp
from jax import lax
from jax.experimental import pallas as pl
from jax.experimental.pallas import tpu as pltpu
from jax.experimental.pallas import tpu_sc as plsc

N = 10000
D = 128
R = 8
NPAD = 10240
NC, NS = 2, 16
NW = NC * NS
B = 128
ROWBLK = 1000


def _enc_body(x_ref, w_ref, b_ref, rw_ref, h0_ref, z_ref):
    h = jnp.maximum(
        jnp.dot(x_ref[...], w_ref[...], preferred_element_type=jnp.float32)
        + b_ref[...], 0.0)
    h0_ref[...] = h
    for r in range(R):
        z_ref[r] = jnp.dot(h, rw_ref[r], preferred_element_type=jnp.float32)


def _encode(x, W_enc, b_enc, rel_weight):
    nblk = N // ROWBLK
    return pl.pallas_call(
        _enc_body,
        grid=(nblk,),
        in_specs=[
            pl.BlockSpec((ROWBLK, D), lambda i: (i, 0)),
            pl.BlockSpec((D, D), lambda i: (0, 0)),
            pl.BlockSpec((1, D), lambda i: (0, 0)),
            pl.BlockSpec((R, D, D), lambda i: (0, 0, 0)),
        ],
        out_specs=[
            pl.BlockSpec((ROWBLK, D), lambda i: (i, 0)),
            pl.BlockSpec((R, ROWBLK, D), lambda i: (0, i, 0)),
        ],
        out_shape=[
            jax.ShapeDtypeStruct((N, D), jnp.float32),
            jax.ShapeDtypeStruct((R, N, D), jnp.float32),
        ],
    )(x, W_enc, b_enc.reshape(1, D), rel_weight)


def _make_edge_scatter(nb):
    mesh = plsc.VectorSubcoreMesh(core_axis_name="c", subcore_axis_name="s")
    stripe = NPAD // NS

    @functools.partial(
        pl.kernel,
        out_type=jax.ShapeDtypeStruct((NC, NPAD, D), jnp.float32),
        mesh=mesh,
        scratch_types=[
            pltpu.VMEM((B,), jnp.int32),
            pltpu.VMEM((B,), jnp.int32),
            pltpu.VMEM((B,), jnp.int32),
            pltpu.VMEM((B,), jnp.int32),
            pltpu.VMEM((B, D), jnp.float32),
            pltpu.VMEM_SHARED((NPAD, D), jnp.float32),
            pltpu.SemaphoreType.DMA,
        ],
    )
    def k(z_hbm, src_hbm, dst_hbm, et_hbm, zero_hbm, out_hbm,
          src_b, et_b, dst_b, gidx_b, rows, acc, sem):
        c = lax.axis_index("c")
        s = lax.axis_index("s")
        wid = s * NC + c
        pltpu.sync_copy(zero_hbm, acc.at[pl.ds(s * stripe, stripe)])
        plsc.subcore_barrier()
        base = wid * (nb * B)

        def body(b, carry):
            off = base + b * B
            pltpu.sync_copy(src_hbm.at[pl.ds(off, B)], src_b)
            pltpu.sync_copy(et_hbm.at[pl.ds(off, B)], et_b)
            pltpu.sync_copy(dst_hbm.at[pl.ds(off, B)], dst_b)
            for j in range(B // 16):
                sl = pl.ds(j * 16, 16)
                gidx_b[sl] = et_b[sl] * N + src_b[sl]
            pltpu.async_copy(z_hbm.at[gidx_b], rows, sem).wait()
            pltpu.sync_copy(rows, acc.at[dst_b], add=True)
            return carry

        lax.fori_loop(0, nb, body, 0)
        plsc.subcore_barrier()
        pltpu.sync_copy(acc.at[pl.ds(s * stripe, stripe)],
                        out_hbm.at[c, pl.ds(s * stripe, stripe)])

    return k


def _final_body(h0_ref, p0_ref, p1_ref, lw_ref, b_ref, o_ref):
    h0 = h0_ref[...]
    agg = p0_ref[0] + p1_ref[0]
    h1 = jnp.maximum(
        agg + jnp.dot(h0, lw_ref[...], preferred_element_type=jnp.float32)
        + b_ref[...], 0.0)
    o_ref[...] = h0 + h1


def _finalize(h0, P, loop_weight, h_bias):
    nblk = N // ROWBLK
    return pl.pallas_call(
        _final_body,
        grid=(nblk,),
        in_specs=[
            pl.BlockSpec((ROWBLK, D), lambda i: (i, 0)),
            pl.BlockSpec((1, ROWBLK, D), lambda i: (0, i, 0)),
            pl.BlockSpec((1, ROWBLK, D), lambda i: (1, i, 0)),
            pl.BlockSpec((D, D), lambda i: (0, 0)),
            pl.BlockSpec((1, D), lambda i: (0, 0)),
        ],
        out_specs=pl.BlockSpec((ROWBLK, D), lambda i: (i, 0)),
        out_shape=jax.ShapeDtypeStruct((N, D), jnp.float32),
    )(h0, P, P, loop_weight, h_bias.reshape(1, D))


def kernel(edge_index, node_features, edgetypes, W_enc, b_enc,
           rel_weight, loop_weight, h_bias):
    E = edge_index.shape[1]
    h0, Z = _encode(node_features, W_enc, b_enc, rel_weight)
    Z2 = Z.reshape(R * N, D)

    per_tile = -(-E // (NW * B)) * B
    e_pad = per_tile * NW
    pad = e_pad - E
    src = jnp.concatenate([edge_index[0], jnp.zeros((pad,), jnp.int32)])
    dst = jnp.concatenate([edge_index[1],
                           jnp.full((pad,), NPAD - 1, jnp.int32)])
    et = jnp.concatenate([edgetypes, jnp.zeros((pad,), jnp.int32)])
    zeros = jnp.zeros((NPAD // NS, D), jnp.float32)

    P = _make_edge_scatter(per_tile // B)(Z2, src, dst, et, zeros)
    return _finalize(h0, P, loop_weight, h_bias)


# speedup vs baseline: 16.8120x; 1.1249x over previous
import functools

import jax
import jax.numpy as jnp
from jax import lax
from jax.experimental import pallas as pl
from jax.experimental.pallas import tpu as pltpu
from jax.experimental.pallas import tpu_sc as plsc

N = 10000
D = 128
R = 8
NPAD = 10240
NC, NS = 2, 16
NW = NC * NS
B = 128
ROWBLK = 1000


def _enc_body(x_ref, w_ref, b_ref, rw_ref, h0_ref, z_ref):
    h = jnp.maximum(
        jnp.dot(x_ref[...], w_ref[...], preferred_element_type=jnp.float32)
        + b_ref[...], 0.0)
    h0_ref[...] = h
    for r in range(R):
        zr = jnp.dot(h, rw_ref[r], preferred_element_type=jnp.float32)
        z_ref[0, r] = zr[:, :D // 2]
        z_ref[1, r] = zr[:, D // 2:]


def _encode(x, W_enc, b_enc, rel_weight):
    nblk = N // ROWBLK
    return pl.pallas_call(
        _enc_body,
        grid=(nblk,),
        in_specs=[
            pl.BlockSpec((ROWBLK, D), lambda i: (i, 0)),
            pl.BlockSpec((D, D), lambda i: (0, 0)),
            pl.BlockSpec((1, D), lambda i: (0, 0)),
            pl.BlockSpec((R, D, D), lambda i: (0, 0, 0)),
        ],
        out_specs=[
            pl.BlockSpec((ROWBLK, D), lambda i: (i, 0)),
            pl.BlockSpec((2, R, ROWBLK, D // 2), lambda i: (0, 0, i, 0)),
        ],
        out_shape=[
            jax.ShapeDtypeStruct((N, D), jnp.float32),
            jax.ShapeDtypeStruct((2, R, N, D // 2), jnp.float32),
        ],
    )(x, W_enc, b_enc.reshape(1, D), rel_weight)


def _gidx_body(src_ref, et_ref, g_ref):
    g_ref[...] = et_ref[...] * N + src_ref[...]


def _make_gidx(rows_, cols):
    return pl.pallas_call(
        _gidx_body,
        out_shape=jax.ShapeDtypeStruct((rows_, cols), jnp.int32),
    )


def _make_edge_scatter(nb):
    mesh = plsc.VectorSubcoreMesh(core_axis_name="c", subcore_axis_name="s")
    stripe = NPAD // NS
    H = D // 2

    @functools.partial(
        pl.kernel,
        out_type=jax.ShapeDtypeStruct((NC, NPAD, H), jnp.float32),
        mesh=mesh,
        compiler_params=pltpu.CompilerParams(use_tc_tiling_on_sc=False),
        scratch_types=[
            pltpu.VMEM((nb, B), jnp.int32),
            pltpu.VMEM((nb, B), jnp.int32),
            pltpu.VMEM((4, B, H), jnp.float32),
            pltpu.VMEM_SHARED((NPAD, H), jnp.float32),
            pltpu.SemaphoreType.DMA,
            pltpu.SemaphoreType.DMA,
            pltpu.SemaphoreType.DMA,
            pltpu.SemaphoreType.DMA,
        ],
    )
    def k(z_hbm, gidx_hbm, dst_hbm, zero_hbm, out_hbm,
          gidx_v, dst_v, ring, acc, sem0, sem1, sem2, sem3):
        c = lax.axis_index("c")
        s = lax.axis_index("s")
        sems = (sem0, sem1, sem2, sem3)
        pltpu.sync_copy(gidx_hbm.at[s], gidx_v)
        pltpu.sync_copy(dst_hbm.at[s], dst_v)
        pltpu.sync_copy(zero_hbm, acc.at[pl.ds(s * stripe, stripe)])
        plsc.subcore_barrier()

        def fire(b, j):
            pltpu.async_copy(z_hbm.at[c].at[gidx_v.at[b]], ring.at[j], sems[j])

        def drain(b, j):
            pltpu.make_async_copy(z_hbm.at[c].at[gidx_v.at[b]], ring.at[j],
                                  sems[j]).wait()

        def scat(b, j):
            pltpu.sync_copy(ring.at[j], acc.at[dst_v.at[b]], add=True)

        for j in range(4):
            fire(j, j)

        def body(i, carry):
            b = 4 * i
            for j in range(4):
                drain(b + j, j)
                fire(b + 4 + j, j)
                scat(b + j, j)
            return carry

        lax.fori_loop(0, nb // 4 - 1, body, 0)
        b = nb - 4
        for j in range(4):
            drain(b + j, j)
            scat(b + j, j)

        plsc.subcore_barrier()
        pltpu.sync_copy(acc.at[pl.ds(s * stripe, stripe)],
                        out_hbm.at[c, pl.ds(s * stripe, stripe)])

    return k


def _final_body(h0_ref, p0_ref, p1_ref, lw_ref, b_ref, o_ref):
    h0 = h0_ref[...]
    agg = jnp.concatenate([p0_ref[0], p1_ref[0]], axis=-1)
    h1 = jnp.maximum(
        agg + jnp.dot(h0, lw_ref[...], preferred_element_type=jnp.float32)
        + b_ref[...], 0.0)
    o_ref[...] = h0 + h1


def _finalize(h0, P, loop_weight, h_bias):
    nblk = N // ROWBLK
    return pl.pallas_call(
        _final_body,
        grid=(nblk,),
        in_specs=[
            pl.BlockSpec((ROWBLK, D), lambda i: (i, 0)),
            pl.BlockSpec((1, ROWBLK, D // 2), lambda i: (0, i, 0)),
            pl.BlockSpec((1, ROWBLK, D // 2), lambda i: (1, i, 0)),
            pl.BlockSpec((D, D), lambda i: (0, 0)),
            pl.BlockSpec((1, D), lambda i: (0, 0)),
        ],
        out_specs=pl.BlockSpec((ROWBLK, D), lambda i: (i, 0)),
        out_shape=jax.ShapeDtypeStruct((N, D), jnp.float32),
    )(h0, P, P, loop_weight, h_bias.reshape(1, D))


def kernel(edge_index, node_features, edgetypes, W_enc, b_enc,
           rel_weight, loop_weight, h_bias):
    E = edge_index.shape[1]
    h0, Z = _encode(node_features, W_enc, b_enc, rel_weight)
    Z2 = Z.reshape(NC, R * N, D // 2)

    per_tile = -(-E // (NS * 4 * B)) * 4 * B
    e_pad = per_tile * NS
    pad = e_pad - E
    src = jnp.concatenate([edge_index[0], jnp.zeros((pad,), jnp.int32)])
    dst = jnp.concatenate([edge_index[1],
                           jnp.full((pad,), NPAD - 1, jnp.int32)])
    et = jnp.concatenate([edgetypes, jnp.zeros((pad,), jnp.int32)])
    zeros = jnp.zeros((NPAD // NS, D // 2), jnp.float32)

    nb = per_tile // B
    gidx = _make_gidx(e_pad // 512, 512)(src.reshape(e_pad // 512, 512),
                                         et.reshape(e_pad // 512, 512))
    gidx3 = gidx.reshape(NS, nb, B)
    dst3 = dst.reshape(NS, nb, B)
    P = _make_edge_scatter(nb)(Z2, gidx3, dst3, zeros)
    return _finalize(h0, P, loop_weight, h_bias)
